# baseline (speedup 1.0000x reference)
import functools

import jax
import jax.numpy as jnp
from jax import lax
from jax.experimental import pallas as pl
from jax.experimental.pallas import tpu as pltpu
from jax.experimental.pallas import tpu_sc as plsc

NUM_TEAMS = 100000
EMBED_DIM = 128
INPUT_DIM = 256
HIDDEN_DIM = 1024
OUTPUT_DIM = 64
BATCH = 16384

NC = 2
NS = 16
NW = NC * NS
CHUNKS = 2
CHUNK_B = BATCH // CHUNKS
B_PER_W = CHUNK_B // NW
IDX_CHUNK = 128
N_IDX_CHUNKS = B_PER_W // IDX_CHUNK


def _sc_gather_body(home_hbm, away_hbm, idx_h_hbm, idx_a_hbm,
                    home_out, away_out, idx_v, rows_v, sem):
  wid = lax.axis_index("s") * NC + lax.axis_index("c")
  base = wid * B_PER_W

  def one_table(table_hbm, idx_hbm, out_hbm):
    pltpu.sync_copy(idx_hbm.at[pl.ds(base, B_PER_W)], idx_v)
    copies = []
    for j in range(N_IDX_CHUNKS):
      copies.append(pltpu.async_copy(
          table_hbm.at[idx_v.at[pl.ds(j * IDX_CHUNK, IDX_CHUNK)]],
          rows_v.at[pl.ds(j * IDX_CHUNK, IDX_CHUNK)], sem))
    for c in copies:
      c.wait()
    pltpu.sync_copy(rows_v, out_hbm.at[pl.ds(base, B_PER_W)])

  one_table(home_hbm, idx_h_hbm, home_out)
  one_table(away_hbm, idx_a_hbm, away_out)


@functools.cache
def _get_sc_gather():
  return pl.kernel(
      _sc_gather_body,
      out_type=(
          jax.ShapeDtypeStruct((CHUNK_B, EMBED_DIM), jnp.float32),
          jax.ShapeDtypeStruct((CHUNK_B, EMBED_DIM), jnp.float32),
      ),
      mesh=plsc.VectorSubcoreMesh(core_axis_name="c", subcore_axis_name="s"),
      scratch_types=[
          pltpu.VMEM((B_PER_W,), jnp.int32),
          pltpu.VMEM((B_PER_W, EMBED_DIM), jnp.float32),
          pltpu.SemaphoreType.DMA,
      ],
  )


BM = 1024


def _mlp_body(home_ref, away_ref, xo_ref, w1_ref, b1_ref, w2_ref, b2_ref,
              out_ref):
  bf = jnp.bfloat16
  x = jnp.concatenate([home_ref[...].astype(bf), away_ref[...].astype(bf),
                       xo_ref[...].astype(bf)], axis=1)
  acc = jnp.dot(x, w1_ref[...], preferred_element_type=jnp.float32)
  h = jnp.maximum(acc + b1_ref[...], 0.0)
  out_ref[...] = jnp.dot(h.astype(bf), w2_ref[...],
                         preferred_element_type=jnp.float32) + b2_ref[...]


def _mlp(home, away, x_other, w1, b1, w2, b2):
  grid = (CHUNK_B // BM,)
  return pl.pallas_call(
      _mlp_body,
      grid=grid,
      in_specs=[
          pl.BlockSpec((BM, EMBED_DIM), lambda i: (i, 0)),
          pl.BlockSpec((BM, EMBED_DIM), lambda i: (i, 0)),
          pl.BlockSpec((BM, INPUT_DIM), lambda i: (i, 0)),
          pl.BlockSpec((2 * EMBED_DIM + INPUT_DIM, HIDDEN_DIM),
                       lambda i: (0, 0)),
          pl.BlockSpec((1, HIDDEN_DIM), lambda i: (0, 0)),
          pl.BlockSpec((HIDDEN_DIM, OUTPUT_DIM), lambda i: (0, 0)),
          pl.BlockSpec((1, OUTPUT_DIM), lambda i: (0, 0)),
      ],
      out_specs=pl.BlockSpec((BM, OUTPUT_DIM), lambda i: (i, 0)),
      out_shape=jax.ShapeDtypeStruct((CHUNK_B, OUTPUT_DIM), jnp.float32),
      compiler_params=pltpu.CompilerParams(
          dimension_semantics=("arbitrary",),
      ),
  )(home, away, x_other, w1, b1, w2, b2)


@jax.jit
def kernel(x_teams, x_other, home_table, away_table, W1, b1, W2, b2):
  idx_home = x_teams[:, 0]
  idx_away = x_teams[:, 1]
  w1 = W1.astype(jnp.bfloat16)
  w2 = W2.astype(jnp.bfloat16)
  b1r = b1.reshape(1, HIDDEN_DIM)
  b2r = b2.reshape(1, OUTPUT_DIM)
  sc = _get_sc_gather()

  gathered = []
  for c in range(CHUNKS):
    lo, hi = c * CHUNK_B, (c + 1) * CHUNK_B
    gathered.append(sc(home_table, away_table,
                       idx_home[lo:hi], idx_away[lo:hi]))
  outs = []
  for c in range(CHUNKS):
    home_rows, away_rows = gathered[c]
    lo, hi = c * CHUNK_B, (c + 1) * CHUNK_B
    outs.append(_mlp(home_rows, away_rows, x_other[lo:hi],
                     w1, b1r, w2, b2r))
  return jnp.concatenate(outs, axis=0)


# speedup vs baseline: 1.1613x; 1.1613x over previous
import functools

import jax
import jax.numpy as jnp
from jax import lax
from jax.experimental import pallas as pl
from jax.experimental.pallas import tpu as pltpu
from jax.experimental.pallas import tpu_sc as plsc

NUM_TEAMS = 100000
EMBED_DIM = 128
INPUT_DIM = 256
HIDDEN_DIM = 1024
OUTPUT_DIM = 64
BATCH = 16384

NC = 2
NS = 16
NW = NC * NS
CHUNKS = 2
CHUNK_B = BATCH // CHUNKS
B_PER_W = CHUNK_B // NW
IDX_CHUNK = 128
N_IDX_CHUNKS = B_PER_W // IDX_CHUNK


def _sc_gather_body(home_hbm, away_hbm, idx_h_hbm, idx_a_hbm,
                    home_out, away_out, idx_v, rows_v, sem):
  wid = lax.axis_index("s") * NC + lax.axis_index("c")
  base = wid * B_PER_W

  def one_table(table_hbm, idx_hbm, out_hbm):
    pltpu.sync_copy(idx_hbm.at[pl.ds(base, B_PER_W)], idx_v)
    copies = []
    for j in range(N_IDX_CHUNKS):
      copies.append(pltpu.async_copy(
          table_hbm.at[idx_v.at[pl.ds(j * IDX_CHUNK, IDX_CHUNK)]],
          rows_v.at[pl.ds(j * IDX_CHUNK, IDX_CHUNK)], sem))
    for c in copies:
      c.wait()
    pltpu.sync_copy(rows_v, out_hbm.at[pl.ds(base, B_PER_W)])

  one_table(home_hbm, idx_h_hbm, home_out)
  one_table(away_hbm, idx_a_hbm, away_out)


@functools.cache
def _get_sc_gather():
  return pl.kernel(
      _sc_gather_body,
      out_type=(
          jax.ShapeDtypeStruct((CHUNK_B, EMBED_DIM), jnp.float32),
          jax.ShapeDtypeStruct((CHUNK_B, EMBED_DIM), jnp.float32),
      ),
      mesh=plsc.VectorSubcoreMesh(core_axis_name="c", subcore_axis_name="s"),
      scratch_types=[
          pltpu.VMEM((B_PER_W,), jnp.int32),
          pltpu.VMEM((B_PER_W, EMBED_DIM), jnp.float32),
          pltpu.SemaphoreType.DMA,
      ],
  )


BM = 1024


def _mlp_body(home_ref, away_ref, xo_ref, w1_ref, b1_ref, w2_ref, b2_ref,
              out_ref):
  bf = jnp.bfloat16
  x = jnp.concatenate([home_ref[...].astype(bf), away_ref[...].astype(bf),
                       xo_ref[...].astype(bf)], axis=1)
  acc = jnp.dot(x, w1_ref[...], preferred_element_type=jnp.float32)
  h = jnp.maximum(acc + b1_ref[...], 0.0)
  out_ref[...] = jnp.dot(h.astype(bf), w2_ref[...],
                         preferred_element_type=jnp.float32) + b2_ref[...]


def _mlp_chunk_body(home_ref, away_ref, xo_ref, w1_ref, b1_ref, w2_ref,
                    b2_ref, prev_ref, out_ref):
  del prev_ref
  _mlp_body(home_ref, away_ref, xo_ref, w1_ref, b1_ref, w2_ref, b2_ref,
            out_ref)


def _mlp_chunk(chunk, home, away, x_other, w1, b1, w2, b2, prev_out):
  off = chunk * (CHUNK_B // BM)
  return pl.pallas_call(
      _mlp_chunk_body,
      grid=(CHUNK_B // BM,),
      in_specs=[
          pl.BlockSpec((BM, EMBED_DIM), lambda i: (i, 0)),
          pl.BlockSpec((BM, EMBED_DIM), lambda i: (i, 0)),
          pl.BlockSpec((BM, INPUT_DIM), lambda i: (i + off, 0)),
          pl.BlockSpec((2 * EMBED_DIM + INPUT_DIM, HIDDEN_DIM),
                       lambda i: (0, 0)),
          pl.BlockSpec((1, HIDDEN_DIM), lambda i: (0, 0)),
          pl.BlockSpec((HIDDEN_DIM, OUTPUT_DIM), lambda i: (0, 0)),
          pl.BlockSpec((1, OUTPUT_DIM), lambda i: (0, 0)),
          pl.BlockSpec(memory_space=pl.ANY),
      ],
      out_specs=pl.BlockSpec((BM, OUTPUT_DIM), lambda i: (i + off, 0)),
      out_shape=jax.ShapeDtypeStruct((BATCH, OUTPUT_DIM), jnp.float32),
      input_output_aliases={7: 0},
      compiler_params=pltpu.CompilerParams(
          dimension_semantics=("arbitrary",),
      ),
  )(home, away, x_other, w1, b1, w2, b2, prev_out)


@jax.jit
def kernel(x_teams, x_other, home_table, away_table, W1, b1, W2, b2):
  idx_home = x_teams[:, 0]
  idx_away = x_teams[:, 1]
  w1 = W1.astype(jnp.bfloat16)
  w2 = W2.astype(jnp.bfloat16)
  b1r = b1.reshape(1, HIDDEN_DIM)
  b2r = b2.reshape(1, OUTPUT_DIM)
  sc = _get_sc_gather()

  gathered = []
  for c in range(CHUNKS):
    lo, hi = c * CHUNK_B, (c + 1) * CHUNK_B
    gathered.append(sc(home_table, away_table,
                       idx_home[lo:hi], idx_away[lo:hi]))
  out = jnp.zeros((BATCH, OUTPUT_DIM), jnp.float32)
  for c in range(CHUNKS):
    home_rows, away_rows = gathered[c]
    out = _mlp_chunk(c, home_rows, away_rows, x_other, w1, b1r, w2, b2r, out)
  return out


# speedup vs baseline: 1.3824x; 1.1903x over previous
import functools

import jax
import jax.numpy as jnp
from jax import lax
from jax.experimental import pallas as pl
from jax.experimental.pallas import tpu as pltpu
from jax.experimental.pallas import tpu_sc as plsc

NUM_TEAMS = 100000
EMBED_DIM = 128
INPUT_DIM = 256
HIDDEN_DIM = 1024
OUTPUT_DIM = 64
BATCH = 16384

NC = 2
NS = 16
NW = NC * NS
CHUNKS = 2
CHUNK_B = BATCH // CHUNKS
B_PER_W = CHUNK_B // NW
IDX_CHUNK = 128
N_IDX_CHUNKS = B_PER_W // IDX_CHUNK


def _sc_gather_body(home_hbm, away_hbm, idx_h_hbm, idx_a_hbm,
                    home_out, away_out, idx_v, rows_v, sem):
  wid = lax.axis_index("s") * NC + lax.axis_index("c")
  base = wid * B_PER_W

  def one_table(table_hbm, idx_hbm, out_hbm):
    pltpu.sync_copy(idx_hbm.at[pl.ds(base, B_PER_W)], idx_v)
    copies = []
    for j in range(N_IDX_CHUNKS):
      copies.append(pltpu.async_copy(
          table_hbm.at[idx_v.at[pl.ds(j * IDX_CHUNK, IDX_CHUNK)]],
          rows_v.at[pl.ds(j * IDX_CHUNK, IDX_CHUNK)], sem))
    for c in copies:
      c.wait()
    pltpu.sync_copy(rows_v, out_hbm.at[pl.ds(base, B_PER_W)])

  one_table(home_hbm, idx_h_hbm, home_out)
  one_table(away_hbm, idx_a_hbm, away_out)


@functools.cache
def _get_sc_gather():
  return pl.kernel(
      _sc_gather_body,
      out_type=(
          jax.ShapeDtypeStruct((CHUNK_B, EMBED_DIM), jnp.float32),
          jax.ShapeDtypeStruct((CHUNK_B, EMBED_DIM), jnp.float32),
      ),
      mesh=plsc.VectorSubcoreMesh(core_axis_name="c", subcore_axis_name="s"),
      scratch_types=[
          pltpu.VMEM((B_PER_W,), jnp.int32),
          pltpu.VMEM((B_PER_W, EMBED_DIM), jnp.float32),
          pltpu.SemaphoreType.DMA,
      ],
  )


BM = 1024


def _mlp_body(home_ref, away_ref, xo_ref, w1_ref, b1_ref, w2t_ref, b2_ref,
              out_ref):
  bf = jnp.bfloat16
  x = jnp.concatenate([home_ref[...].astype(bf), away_ref[...].astype(bf),
                       xo_ref[...].astype(bf)], axis=1)
  acc = jnp.dot(x, w1_ref[...], preferred_element_type=jnp.float32)
  h = jnp.maximum(acc + b1_ref[...], 0.0)
  out_t = lax.dot_general(w2t_ref[...], h.astype(bf),
                          (((1,), (1,)), ((), ())),
                          preferred_element_type=jnp.float32)
  out_ref[...] = out_t + b2_ref[...]


def _mlp_chunk_body(home_ref, away_ref, xo_ref, w1_ref, b1_ref, w2t_ref,
                    b2_ref, prev_ref, out_ref):
  del prev_ref
  _mlp_body(home_ref, away_ref, xo_ref, w1_ref, b1_ref, w2t_ref, b2_ref,
            out_ref)


def _mlp_chunk(chunk, home, away, x_other, w1, b1, w2t, b2, prev_out):
  off = chunk * (CHUNK_B // BM)
  body = _mlp_body if prev_out is None else _mlp_chunk_body
  in_specs = [
      pl.BlockSpec((BM, EMBED_DIM), lambda i: (i, 0)),
      pl.BlockSpec((BM, EMBED_DIM), lambda i: (i, 0)),
      pl.BlockSpec((BM, INPUT_DIM), lambda i: (i + off, 0)),
      pl.BlockSpec((2 * EMBED_DIM + INPUT_DIM, HIDDEN_DIM),
                   lambda i: (0, 0)),
      pl.BlockSpec((1, HIDDEN_DIM), lambda i: (0, 0)),
      pl.BlockSpec((OUTPUT_DIM, HIDDEN_DIM), lambda i: (0, 0)),
      pl.BlockSpec((OUTPUT_DIM, 1), lambda i: (0, 0)),
  ]
  args = [home, away, x_other, w1, b1, w2t, b2]
  aliases = {}
  if prev_out is not None:
    in_specs.append(pl.BlockSpec(memory_space=pl.ANY))
    args.append(prev_out)
    aliases = {7: 0}
  return pl.pallas_call(
      body,
      grid=(CHUNK_B // BM,),
      in_specs=in_specs,
      out_specs=pl.BlockSpec((OUTPUT_DIM, BM), lambda i: (0, i + off)),
      out_shape=jax.ShapeDtypeStruct((OUTPUT_DIM, BATCH), jnp.float32),
      input_output_aliases=aliases,
      compiler_params=pltpu.CompilerParams(
          dimension_semantics=("arbitrary",),
      ),
  )(*args)


@jax.jit
def kernel(x_teams, x_other, home_table, away_table, W1, b1, W2, b2):
  idx_home = x_teams[:, 0]
  idx_away = x_teams[:, 1]
  w1 = W1.astype(jnp.bfloat16)
  w2t = W2.T.astype(jnp.bfloat16)
  b1r = b1.reshape(1, HIDDEN_DIM)
  b2r = b2.reshape(OUTPUT_DIM, 1)
  sc = _get_sc_gather()

  gathered = []
  for c in range(CHUNKS):
    lo, hi = c * CHUNK_B, (c + 1) * CHUNK_B
    gathered.append(sc(home_table, away_table,
                       idx_home[lo:hi], idx_away[lo:hi]))
  out = None
  for c in range(CHUNKS):
    home_rows, away_rows = gathered[c]
    out = _mlp_chunk(c, home_rows, away_rows, x_other, w1, b1r, w2t, b2r, out)
  return out.T


# speedup vs baseline: 1.4022x; 1.0143x over previous
import functools

import jax
import jax.numpy as jnp
from jax import lax
from jax.experimental import pallas as pl
from jax.experimental.pallas import tpu as pltpu
from jax.experimental.pallas import tpu_sc as plsc

NUM_TEAMS = 100000
EMBED_DIM = 128
INPUT_DIM = 256
HIDDEN_DIM = 1024
OUTPUT_DIM = 64
BATCH = 16384

NC = 2
NS = 16
NW = NC * NS
CHUNKS = 2
CHUNK_B = BATCH // CHUNKS
B_PER_W = CHUNK_B // NW
IDX_CHUNK = 128
N_IDX_CHUNKS = B_PER_W // IDX_CHUNK


def _sc_gather_body(home_hbm, away_hbm, idx_h_hbm, idx_a_hbm,
                    home_out, away_out, idx_v, rows_v, sem):
  wid = lax.axis_index("s") * NC + lax.axis_index("c")
  base = wid * B_PER_W

  def one_table(table_hbm, idx_hbm, out_hbm):
    pltpu.sync_copy(idx_hbm.at[pl.ds(base, B_PER_W)], idx_v)
    copies = []
    for j in range(N_IDX_CHUNKS):
      copies.append(pltpu.async_copy(
          table_hbm.at[idx_v.at[pl.ds(j * IDX_CHUNK, IDX_CHUNK)]],
          rows_v.at[pl.ds(j * IDX_CHUNK, IDX_CHUNK)], sem))
    for c in copies:
      c.wait()
    pltpu.sync_copy(rows_v, out_hbm.at[pl.ds(base, B_PER_W)])

  one_table(home_hbm, idx_h_hbm, home_out)
  one_table(away_hbm, idx_a_hbm, away_out)


@functools.cache
def _get_sc_gather():
  return pl.kernel(
      _sc_gather_body,
      out_type=(
          jax.ShapeDtypeStruct((CHUNK_B, EMBED_DIM), jnp.float32),
          jax.ShapeDtypeStruct((CHUNK_B, EMBED_DIM), jnp.float32),
      ),
      mesh=plsc.VectorSubcoreMesh(core_axis_name="c", subcore_axis_name="s"),
      scratch_types=[
          pltpu.VMEM((B_PER_W,), jnp.int32),
          pltpu.VMEM((B_PER_W, EMBED_DIM), jnp.float32),
          pltpu.SemaphoreType.DMA,
      ],
  )


BM = 2048


def _mlp_body(home_ref, away_ref, xo_ref, w1_ref, b1_ref, w2t_ref, b2_ref,
              out_ref):
  bf = jnp.bfloat16
  x = jnp.concatenate([home_ref[...].astype(bf), away_ref[...].astype(bf),
                       xo_ref[...].astype(bf)], axis=1)
  acc = jnp.dot(x, w1_ref[...], preferred_element_type=jnp.float32)
  h = jnp.maximum(acc + b1_ref[...], 0.0)
  out_t = lax.dot_general(w2t_ref[...], h.astype(bf),
                          (((1,), (1,)), ((), ())),
                          preferred_element_type=jnp.float32)
  out_ref[...] = out_t + b2_ref[...]


def _mlp_chunk_body(home_ref, away_ref, xo_ref, w1_ref, b1_ref, w2t_ref,
                    b2_ref, prev_ref, out_ref):
  del prev_ref
  _mlp_body(home_ref, away_ref, xo_ref, w1_ref, b1_ref, w2t_ref, b2_ref,
            out_ref)


def _mlp_chunk(chunk, home, away, x_other, w1, b1, w2t, b2, prev_out):
  off = chunk * (CHUNK_B // BM)
  body = _mlp_body if prev_out is None else _mlp_chunk_body
  in_specs = [
      pl.BlockSpec((BM, EMBED_DIM), lambda i: (i, 0)),
      pl.BlockSpec((BM, EMBED_DIM), lambda i: (i, 0)),
      pl.BlockSpec((BM, INPUT_DIM), lambda i: (i + off, 0)),
      pl.BlockSpec((2 * EMBED_DIM + INPUT_DIM, HIDDEN_DIM),
                   lambda i: (0, 0)),
      pl.BlockSpec((1, HIDDEN_DIM), lambda i: (0, 0)),
      pl.BlockSpec((OUTPUT_DIM, HIDDEN_DIM), lambda i: (0, 0)),
      pl.BlockSpec((OUTPUT_DIM, 1), lambda i: (0, 0)),
  ]
  args = [home, away, x_other, w1, b1, w2t, b2]
  aliases = {}
  if prev_out is not None:
    in_specs.append(pl.BlockSpec(memory_space=pl.ANY))
    args.append(prev_out)
    aliases = {7: 0}
  return pl.pallas_call(
      body,
      grid=(CHUNK_B // BM,),
      in_specs=in_specs,
      out_specs=pl.BlockSpec((OUTPUT_DIM, BM), lambda i: (0, i + off)),
      out_shape=jax.ShapeDtypeStruct((OUTPUT_DIM, BATCH), jnp.float32),
      input_output_aliases=aliases,
      compiler_params=pltpu.CompilerParams(
          dimension_semantics=("arbitrary",),
      ),
  )(*args)


@jax.jit
def kernel(x_teams, x_other, home_table, away_table, W1, b1, W2, b2):
  idx_home = x_teams[:, 0]
  idx_away = x_teams[:, 1]
  w1 = W1.astype(jnp.bfloat16)
  w2t = W2.T.astype(jnp.bfloat16)
  b1r = b1.reshape(1, HIDDEN_DIM)
  b2r = b2.reshape(OUTPUT_DIM, 1)
  sc = _get_sc_gather()

  gathered = []
  for c in range(CHUNKS):
    lo, hi = c * CHUNK_B, (c + 1) * CHUNK_B
    gathered.append(sc(home_table, away_table,
                       idx_home[lo:hi], idx_away[lo:hi]))
  out = None
  for c in range(CHUNKS):
    home_rows, away_rows = gathered[c]
    out = _mlp_chunk(c, home_rows, away_rows, x_other, w1, b1r, w2t, b2r, out)
  return out.T


# speedup vs baseline: 1.4046x; 1.0018x over previous
import functools

import jax
import jax.numpy as jnp
from jax import lax
from jax.experimental import pallas as pl
from jax.experimental.pallas import tpu as pltpu
from jax.experimental.pallas import tpu_sc as plsc

NUM_TEAMS = 100000
EMBED_DIM = 128
INPUT_DIM = 256
HIDDEN_DIM = 1024
OUTPUT_DIM = 64
BATCH = 16384

NC = 2
NS = 16
NW = NC * NS
CHUNKS = 2
CHUNK_B = BATCH // CHUNKS
B_PER_W = CHUNK_B // NW
IDX_CHUNK = 128
N_IDX_CHUNKS = B_PER_W // IDX_CHUNK


def _sc_gather_body(chunk_base, home_hbm, away_hbm, idx_h_hbm, idx_a_hbm,
                    home_out, away_out, idx_v, rows_v, sem):
  wid = lax.axis_index("s") * NC + lax.axis_index("c")
  base = wid * B_PER_W

  def one_table(table_hbm, idx_hbm, out_hbm):
    pltpu.sync_copy(idx_hbm.at[pl.ds(chunk_base + base, B_PER_W)], idx_v)
    copies = []
    for j in range(N_IDX_CHUNKS):
      copies.append(pltpu.async_copy(
          table_hbm.at[idx_v.at[pl.ds(j * IDX_CHUNK, IDX_CHUNK)]],
          rows_v.at[pl.ds(j * IDX_CHUNK, IDX_CHUNK)], sem))
    for c in copies:
      c.wait()
    pltpu.sync_copy(rows_v, out_hbm.at[pl.ds(base, B_PER_W)])

  one_table(home_hbm, idx_h_hbm, home_out)
  one_table(away_hbm, idx_a_hbm, away_out)


@functools.cache
def _get_sc_gather(chunk_base):
  return pl.kernel(
      functools.partial(_sc_gather_body, chunk_base),
      out_type=(
          jax.ShapeDtypeStruct((CHUNK_B, EMBED_DIM), jnp.float32),
          jax.ShapeDtypeStruct((CHUNK_B, EMBED_DIM), jnp.float32),
      ),
      mesh=plsc.VectorSubcoreMesh(core_axis_name="c", subcore_axis_name="s"),
      scratch_types=[
          pltpu.VMEM((B_PER_W,), jnp.int32),
          pltpu.VMEM((B_PER_W, EMBED_DIM), jnp.float32),
          pltpu.SemaphoreType.DMA,
      ],
  )


BM = 2048


def _mlp_body(home_ref, away_ref, xo_ref, w1_ref, b1_ref, w2t_ref, b2_ref,
              out_ref):
  bf = jnp.bfloat16
  x = jnp.concatenate([home_ref[...].astype(bf), away_ref[...].astype(bf),
                       xo_ref[...].astype(bf)], axis=1)
  acc = jnp.dot(x, w1_ref[...], preferred_element_type=jnp.float32)
  h = jnp.maximum(acc + b1_ref[...], 0.0)
  out_t = lax.dot_general(w2t_ref[...], h.astype(bf),
                          (((1,), (1,)), ((), ())),
                          preferred_element_type=jnp.float32)
  out_ref[...] = out_t + b2_ref[...]


def _mlp_chunk_body(home_ref, away_ref, xo_ref, w1_ref, b1_ref, w2t_ref,
                    b2_ref, prev_ref, out_ref):
  del prev_ref
  _mlp_body(home_ref, away_ref, xo_ref, w1_ref, b1_ref, w2t_ref, b2_ref,
            out_ref)


def _mlp_chunk(chunk, home, away, x_other, w1, b1, w2t, b2, prev_out):
  off = chunk * (CHUNK_B // BM)
  body = _mlp_body if prev_out is None else _mlp_chunk_body
  in_specs = [
      pl.BlockSpec((BM, EMBED_DIM), lambda i: (i, 0)),
      pl.BlockSpec((BM, EMBED_DIM), lambda i: (i, 0)),
      pl.BlockSpec((BM, INPUT_DIM), lambda i: (i + off, 0)),
      pl.BlockSpec((2 * EMBED_DIM + INPUT_DIM, HIDDEN_DIM),
                   lambda i: (0, 0)),
      pl.BlockSpec((1, HIDDEN_DIM), lambda i: (0, 0)),
      pl.BlockSpec((OUTPUT_DIM, HIDDEN_DIM), lambda i: (0, 0)),
      pl.BlockSpec((OUTPUT_DIM, 1), lambda i: (0, 0)),
  ]
  args = [home, away, x_other, w1, b1, w2t, b2]
  aliases = {}
  if prev_out is not None:
    in_specs.append(pl.BlockSpec(memory_space=pl.ANY))
    args.append(prev_out)
    aliases = {7: 0}
  return pl.pallas_call(
      body,
      grid=(CHUNK_B // BM,),
      in_specs=in_specs,
      out_specs=pl.BlockSpec((OUTPUT_DIM, BM), lambda i: (0, i + off)),
      out_shape=jax.ShapeDtypeStruct((OUTPUT_DIM, BATCH), jnp.float32),
      input_output_aliases=aliases,
      compiler_params=pltpu.CompilerParams(
          dimension_semantics=("arbitrary",),
      ),
  )(*args)


@jax.jit
def kernel(x_teams, x_other, home_table, away_table, W1, b1, W2, b2):
  w1 = W1.astype(jnp.bfloat16)
  w2t = W2.T.astype(jnp.bfloat16)
  b1r = b1.reshape(1, HIDDEN_DIM)
  b2r = b2.reshape(OUTPUT_DIM, 1)

  idx_home = x_teams[:, 0]
  idx_away = x_teams[:, 1]
  gathered = []
  for c in range(CHUNKS):
    sc = _get_sc_gather(c * CHUNK_B)
    gathered.append(sc(home_table, away_table, idx_home, idx_away))
  out = None
  for c in range(CHUNKS):
    home_rows, away_rows = gathered[c]
    out = _mlp_chunk(c, home_rows, away_rows, x_other, w1, b1r, w2t, b2r, out)
  return out.T
